# baseline (speedup 1.0000x reference)
import jax
import jax.numpy as jnp
from jax import lax
from jax.experimental import pallas as pl
from jax.experimental.pallas import tpu as pltpu
from jax.experimental.pallas import tpu_sc as plsc

N = 4096
L = 16
NSUB = 32
CHUNK = N // NSUB
QV = 4
NBLK = N // (QV * L)
NEG_BIG = -1e30


def _sc_body(p_hbm, lab_hbm, sums_hbm, meta_hbm,
             pv, labv, qbuf, rsm, obuf, mbuf):
    c = lax.axis_index("c")
    s = lax.axis_index("s")
    wid = c * 16 + s
    base = wid * CHUNK

    pltpu.sync_copy(p_hbm, pv.at[pl.ds(0, N)])
    pltpu.sync_copy(lab_hbm, labv.at[pl.ds(0, N)])

    neg = jnp.full((L,), NEG_BIG, dtype=jnp.float32)

    def build(jv, carry):
        v = pv[pl.ds(jv * L, L)]
        m = labv[pl.ds(jv * L, L)] == 0
        qbuf[pl.ds(jv * L, L)] = jnp.where(m, v, neg)
        return carry

    lax.fori_loop(0, N // L, build, jnp.int32(0), unroll=4)

    def rowc(g, cnt):
        lv = labv[pl.ds(base + g * L, L)]
        pvv = pv[pl.ds(base + g * L, L)]
        for k in range(L):
            li = lv[k]
            pi = pvv[k]

            @pl.when(li == 1)
            def _(cnt=cnt, pi=pi):
                rsm[cnt] = pi

            cnt = cnt + jnp.where(li == 1, jnp.int32(1), jnp.int32(0))
        return cnt

    n1w = lax.fori_loop(0, CHUNK // L, rowc, jnp.int32(0))

    rsm[n1w] = jnp.float32(-NEG_BIG)
    rsm[n1w + 1] = jnp.float32(-NEG_BIG)
    n_pair = (n1w + 1) // 2

    zero = jnp.zeros((L,), dtype=jnp.float32)

    def qblock(b, gaccs):
        q0 = qbuf[pl.ds(b * (QV * L), L)]
        q1 = qbuf[pl.ds(b * (QV * L) + L, L)]
        q2 = qbuf[pl.ds(b * (QV * L) + 2 * L, L)]
        q3 = qbuf[pl.ds(b * (QV * L) + 3 * L, L)]

        def rows(i, accs):
            b0, b1, b2, b3 = accs
            pi = rsm[2 * i]
            pj = rsm[2 * i + 1]
            b0 = b0 + jnp.maximum(q0 - pi, 0.0)
            b1 = b1 + jnp.maximum(q1 - pi, 0.0)
            b2 = b2 + jnp.maximum(q2 - pi, 0.0)
            b3 = b3 + jnp.maximum(q3 - pi, 0.0)
            b0 = b0 + jnp.maximum(q0 - pj, 0.0)
            b1 = b1 + jnp.maximum(q1 - pj, 0.0)
            b2 = b2 + jnp.maximum(q2 - pj, 0.0)
            b3 = b3 + jnp.maximum(q3 - pj, 0.0)
            return (b0, b1, b2, b3)

        return lax.fori_loop(0, n_pair, rows, gaccs)

    a0, a1, a2, a3 = lax.fori_loop(
        0, NBLK, qblock, (zero, zero, zero, zero))

    obuf[...] = (a0 + a1) + (a2 + a3)
    pltpu.sync_copy(obuf, sums_hbm.at[wid])

    lane = lax.broadcasted_iota(jnp.int32, (L,), 0)
    n0w = jnp.int32(CHUNK) - n1w
    meta = jnp.where(lane == 0, n1w.astype(jnp.float32),
                     jnp.where(lane == 1, n0w.astype(jnp.float32), 0.0))
    mbuf[...] = meta
    pltpu.sync_copy(mbuf, meta_hbm.at[wid])


_mesh = plsc.VectorSubcoreMesh(core_axis_name="c", subcore_axis_name="s")

_pairwise_sc = pl.kernel(
    _sc_body,
    out_type=(jax.ShapeDtypeStruct((NSUB, L), jnp.float32),
              jax.ShapeDtypeStruct((NSUB, L), jnp.float32)),
    mesh=_mesh,
    scratch_types=[
        pltpu.VMEM((N + L,), jnp.float32),
        pltpu.VMEM((N + L,), jnp.int32),
        pltpu.VMEM((N,), jnp.float32),
        pltpu.SMEM((CHUNK + 2,), jnp.float32),
        pltpu.VMEM((L,), jnp.float32),
        pltpu.VMEM((L,), jnp.float32),
    ],
)


ROWS_PER_TILE = 128
NTILES = N // ROWS_PER_TILE


def _tc_body(pc_ref, lc_ref, pr_ref, lr_ref, sum_ref, cnt_ref, acc_ref):
    i = pl.program_id(0)

    @pl.when(i == 0)
    def _():
        acc_ref[...] = jnp.zeros((8, 128), jnp.float32)

    r = jnp.where(lc_ref[...] == 1, pc_ref[...], -NEG_BIG)
    q = jnp.where(lr_ref[...] == 0, pr_ref[...], NEG_BIG)
    contrib = jnp.maximum(q - r, 0.0)
    part = contrib.reshape(16, 8, 32, 128).sum(axis=(0, 2))
    acc_ref[...] += part
    cnt_ref[...] = jnp.sum(lc_ref[...].astype(jnp.float32)).reshape(1, 1, 1)

    @pl.when(i == NTILES - 1)
    def _():
        sum_ref[...] = acc_ref[...]


_pairwise_tc = pl.pallas_call(
    _tc_body,
    grid=(NTILES,),
    in_specs=[
        pl.BlockSpec((ROWS_PER_TILE, 1), lambda i: (i, 0)),
        pl.BlockSpec((ROWS_PER_TILE, 1), lambda i: (i, 0)),
        pl.BlockSpec((1, N), lambda i: (0, 0)),
        pl.BlockSpec((1, N), lambda i: (0, 0)),
    ],
    out_specs=[
        pl.BlockSpec((8, 128), lambda i: (0, 0)),
        pl.BlockSpec((1, 1, 1), lambda i: (i, 0, 0)),
    ],
    out_shape=[
        jax.ShapeDtypeStruct((8, 128), jnp.float32),
        jax.ShapeDtypeStruct((NTILES, 1, 1), jnp.float32),
    ],
    scratch_shapes=[pltpu.VMEM((8, 128), jnp.float32)],
)


@jax.jit
def kernel(prediction, label):
    pcol = prediction.reshape(N, 1)
    lcol = label.reshape(N, 1)
    prow = prediction.reshape(1, N)
    lrow = label.reshape(1, N)
    sums, n1t = _pairwise_tc(pcol, lcol, prow, lrow)
    loss_sum = jnp.sum(sums)
    n1 = jnp.sum(n1t)
    n0 = jnp.float32(N) - n1
    count = n1 * n0
    return jnp.where(count > 0, loss_sum / count, jnp.float32(0.0))


# speedup vs baseline: 1.2506x; 1.2506x over previous
import jax
import jax.numpy as jnp
from jax import lax
from jax.experimental import pallas as pl
from jax.experimental.pallas import tpu as pltpu
from jax.experimental.pallas import tpu_sc as plsc

N = 4096
L = 16
NSUB = 32
CHUNK = N // NSUB
QV = 4
NBLK = N // (QV * L)
NEG_BIG = -1e30


def _sc_body(p_hbm, lab_hbm, sums_hbm, meta_hbm,
             pv, labv, qbuf, rsm, obuf, mbuf):
    c = lax.axis_index("c")
    s = lax.axis_index("s")
    wid = c * 16 + s
    base = wid * CHUNK

    pltpu.sync_copy(p_hbm, pv.at[pl.ds(0, N)])
    pltpu.sync_copy(lab_hbm, labv.at[pl.ds(0, N)])

    neg = jnp.full((L,), NEG_BIG, dtype=jnp.float32)

    def build(jv, carry):
        v = pv[pl.ds(jv * L, L)]
        m = labv[pl.ds(jv * L, L)] == 0
        qbuf[pl.ds(jv * L, L)] = jnp.where(m, v, neg)
        return carry

    lax.fori_loop(0, N // L, build, jnp.int32(0), unroll=4)

    def rowc(g, cnt):
        lv = labv[pl.ds(base + g * L, L)]
        pvv = pv[pl.ds(base + g * L, L)]
        for k in range(L):
            li = lv[k]
            pi = pvv[k]

            @pl.when(li == 1)
            def _(cnt=cnt, pi=pi):
                rsm[cnt] = pi

            cnt = cnt + jnp.where(li == 1, jnp.int32(1), jnp.int32(0))
        return cnt

    n1w = lax.fori_loop(0, CHUNK // L, rowc, jnp.int32(0))

    rsm[n1w] = jnp.float32(-NEG_BIG)
    rsm[n1w + 1] = jnp.float32(-NEG_BIG)
    n_pair = (n1w + 1) // 2

    zero = jnp.zeros((L,), dtype=jnp.float32)

    def qblock(b, gaccs):
        q0 = qbuf[pl.ds(b * (QV * L), L)]
        q1 = qbuf[pl.ds(b * (QV * L) + L, L)]
        q2 = qbuf[pl.ds(b * (QV * L) + 2 * L, L)]
        q3 = qbuf[pl.ds(b * (QV * L) + 3 * L, L)]

        def rows(i, accs):
            b0, b1, b2, b3 = accs
            pi = rsm[2 * i]
            pj = rsm[2 * i + 1]
            b0 = b0 + jnp.maximum(q0 - pi, 0.0)
            b1 = b1 + jnp.maximum(q1 - pi, 0.0)
            b2 = b2 + jnp.maximum(q2 - pi, 0.0)
            b3 = b3 + jnp.maximum(q3 - pi, 0.0)
            b0 = b0 + jnp.maximum(q0 - pj, 0.0)
            b1 = b1 + jnp.maximum(q1 - pj, 0.0)
            b2 = b2 + jnp.maximum(q2 - pj, 0.0)
            b3 = b3 + jnp.maximum(q3 - pj, 0.0)
            return (b0, b1, b2, b3)

        return lax.fori_loop(0, n_pair, rows, gaccs)

    a0, a1, a2, a3 = lax.fori_loop(
        0, NBLK, qblock, (zero, zero, zero, zero))

    obuf[...] = (a0 + a1) + (a2 + a3)
    pltpu.sync_copy(obuf, sums_hbm.at[wid])

    lane = lax.broadcasted_iota(jnp.int32, (L,), 0)
    n0w = jnp.int32(CHUNK) - n1w
    meta = jnp.where(lane == 0, n1w.astype(jnp.float32),
                     jnp.where(lane == 1, n0w.astype(jnp.float32), 0.0))
    mbuf[...] = meta
    pltpu.sync_copy(mbuf, meta_hbm.at[wid])


_mesh = plsc.VectorSubcoreMesh(core_axis_name="c", subcore_axis_name="s")

_pairwise_sc = pl.kernel(
    _sc_body,
    out_type=(jax.ShapeDtypeStruct((NSUB, L), jnp.float32),
              jax.ShapeDtypeStruct((NSUB, L), jnp.float32)),
    mesh=_mesh,
    scratch_types=[
        pltpu.VMEM((N + L,), jnp.float32),
        pltpu.VMEM((N + L,), jnp.int32),
        pltpu.VMEM((N,), jnp.float32),
        pltpu.SMEM((CHUNK + 2,), jnp.float32),
        pltpu.VMEM((L,), jnp.float32),
        pltpu.VMEM((L,), jnp.float32),
    ],
)


ROWS_PER_TILE = 128
NTILES = N // ROWS_PER_TILE


def _tc_body(pc_ref, lc_ref, pr_ref, lr_ref, sum_ref, cnt_ref, acc_ref):
    i = pl.program_id(0)

    @pl.when(i == 0)
    def _():
        acc_ref[...] = jnp.zeros((8, 128), jnp.float32)

    r = jnp.where(lc_ref[...] == 1, pc_ref[...], -NEG_BIG)
    q = jnp.where(lr_ref[...] == 0, pr_ref[...], NEG_BIG)
    rb = jnp.broadcast_to(r, (ROWS_PER_TILE, 128))
    parts = [jnp.zeros((8, 128), jnp.float32) for _ in range(8)]
    for d in range(N // 128):
        cd = jnp.maximum(q[:, d * 128:(d + 1) * 128] - rb, 0.0)
        for a in range(16):
            k = (d * 16 + a) % 8
            parts[k] = parts[k] + cd[a * 8:(a + 1) * 8, :]
    part = (((parts[0] + parts[1]) + (parts[2] + parts[3]))
            + ((parts[4] + parts[5]) + (parts[6] + parts[7])))
    acc_ref[...] += part
    cnt_ref[...] = jnp.sum(lc_ref[...].astype(jnp.float32)).reshape(1, 1, 1)

    @pl.when(i == NTILES - 1)
    def _():
        sum_ref[...] = acc_ref[...]


_pairwise_tc = pl.pallas_call(
    _tc_body,
    grid=(NTILES,),
    in_specs=[
        pl.BlockSpec((ROWS_PER_TILE, 1), lambda i: (i, 0)),
        pl.BlockSpec((ROWS_PER_TILE, 1), lambda i: (i, 0)),
        pl.BlockSpec((1, N), lambda i: (0, 0)),
        pl.BlockSpec((1, N), lambda i: (0, 0)),
    ],
    out_specs=[
        pl.BlockSpec((8, 128), lambda i: (0, 0)),
        pl.BlockSpec((1, 1, 1), lambda i: (i, 0, 0)),
    ],
    out_shape=[
        jax.ShapeDtypeStruct((8, 128), jnp.float32),
        jax.ShapeDtypeStruct((NTILES, 1, 1), jnp.float32),
    ],
    scratch_shapes=[pltpu.VMEM((8, 128), jnp.float32)],
)


@jax.jit
def kernel(prediction, label):
    pcol = prediction.reshape(N, 1)
    lcol = label.reshape(N, 1)
    prow = prediction.reshape(1, N)
    lrow = label.reshape(1, N)
    sums, n1t = _pairwise_tc(pcol, lcol, prow, lrow)
    loss_sum = jnp.sum(sums)
    n1 = jnp.sum(n1t)
    n0 = jnp.float32(N) - n1
    count = n1 * n0
    return jnp.where(count > 0, loss_sum / count, jnp.float32(0.0))


# speedup vs baseline: 1.7138x; 1.3704x over previous
import jax
import jax.numpy as jnp
from jax import lax
from jax.experimental import pallas as pl
from jax.experimental.pallas import tpu as pltpu
from jax.experimental.pallas import tpu_sc as plsc

N = 4096
L = 16
NSUB = 32
CHUNK = N // NSUB
QV = 4
NBLK = N // (QV * L)
NEG_BIG = -1e30


def _sc_body(p_hbm, lab_hbm, sums_hbm, meta_hbm,
             pv, labv, qbuf, rsm, obuf, mbuf):
    c = lax.axis_index("c")
    s = lax.axis_index("s")
    wid = c * 16 + s
    base = wid * CHUNK

    pltpu.sync_copy(p_hbm, pv.at[pl.ds(0, N)])
    pltpu.sync_copy(lab_hbm, labv.at[pl.ds(0, N)])

    neg = jnp.full((L,), NEG_BIG, dtype=jnp.float32)

    def build(jv, carry):
        v = pv[pl.ds(jv * L, L)]
        m = labv[pl.ds(jv * L, L)] == 0
        qbuf[pl.ds(jv * L, L)] = jnp.where(m, v, neg)
        return carry

    lax.fori_loop(0, N // L, build, jnp.int32(0), unroll=4)

    def rowc(g, cnt):
        lv = labv[pl.ds(base + g * L, L)]
        pvv = pv[pl.ds(base + g * L, L)]
        for k in range(L):
            li = lv[k]
            pi = pvv[k]

            @pl.when(li == 1)
            def _(cnt=cnt, pi=pi):
                rsm[cnt] = pi

            cnt = cnt + jnp.where(li == 1, jnp.int32(1), jnp.int32(0))
        return cnt

    n1w = lax.fori_loop(0, CHUNK // L, rowc, jnp.int32(0))

    rsm[n1w] = jnp.float32(-NEG_BIG)
    rsm[n1w + 1] = jnp.float32(-NEG_BIG)
    n_pair = (n1w + 1) // 2

    zero = jnp.zeros((L,), dtype=jnp.float32)

    def qblock(b, gaccs):
        q0 = qbuf[pl.ds(b * (QV * L), L)]
        q1 = qbuf[pl.ds(b * (QV * L) + L, L)]
        q2 = qbuf[pl.ds(b * (QV * L) + 2 * L, L)]
        q3 = qbuf[pl.ds(b * (QV * L) + 3 * L, L)]

        def rows(i, accs):
            b0, b1, b2, b3 = accs
            pi = rsm[2 * i]
            pj = rsm[2 * i + 1]
            b0 = b0 + jnp.maximum(q0 - pi, 0.0)
            b1 = b1 + jnp.maximum(q1 - pi, 0.0)
            b2 = b2 + jnp.maximum(q2 - pi, 0.0)
            b3 = b3 + jnp.maximum(q3 - pi, 0.0)
            b0 = b0 + jnp.maximum(q0 - pj, 0.0)
            b1 = b1 + jnp.maximum(q1 - pj, 0.0)
            b2 = b2 + jnp.maximum(q2 - pj, 0.0)
            b3 = b3 + jnp.maximum(q3 - pj, 0.0)
            return (b0, b1, b2, b3)

        return lax.fori_loop(0, n_pair, rows, gaccs)

    a0, a1, a2, a3 = lax.fori_loop(
        0, NBLK, qblock, (zero, zero, zero, zero))

    obuf[...] = (a0 + a1) + (a2 + a3)
    pltpu.sync_copy(obuf, sums_hbm.at[wid])

    lane = lax.broadcasted_iota(jnp.int32, (L,), 0)
    n0w = jnp.int32(CHUNK) - n1w
    meta = jnp.where(lane == 0, n1w.astype(jnp.float32),
                     jnp.where(lane == 1, n0w.astype(jnp.float32), 0.0))
    mbuf[...] = meta
    pltpu.sync_copy(mbuf, meta_hbm.at[wid])


_mesh = plsc.VectorSubcoreMesh(core_axis_name="c", subcore_axis_name="s")

_pairwise_sc = pl.kernel(
    _sc_body,
    out_type=(jax.ShapeDtypeStruct((NSUB, L), jnp.float32),
              jax.ShapeDtypeStruct((NSUB, L), jnp.float32)),
    mesh=_mesh,
    scratch_types=[
        pltpu.VMEM((N + L,), jnp.float32),
        pltpu.VMEM((N + L,), jnp.int32),
        pltpu.VMEM((N,), jnp.float32),
        pltpu.SMEM((CHUNK + 2,), jnp.float32),
        pltpu.VMEM((L,), jnp.float32),
        pltpu.VMEM((L,), jnp.float32),
    ],
)


ROWS_PER_TILE = 128
NTILES = N // ROWS_PER_TILE


def _tc_body(pc_ref, lc_ref, pr_ref, lr_ref, sum_ref, cnt_ref):
    q = jnp.where(lr_ref[...] == 0, pr_ref[...], NEG_BIG)
    zeros8 = jnp.zeros((8, 128), jnp.float32)

    def step(i, carry):
        parts = list(carry[:8])
        cacc = carry[8]
        lc = lc_ref[pl.ds(i * ROWS_PER_TILE, ROWS_PER_TILE), :]
        pc = pc_ref[pl.ds(i * ROWS_PER_TILE, ROWS_PER_TILE), :]
        r = jnp.where(lc == 1, pc, -NEG_BIG)
        rb = jnp.broadcast_to(r, (ROWS_PER_TILE, 128))
        for d in range(N // 128):
            cd = jnp.maximum(q[:, d * 128:(d + 1) * 128] - rb, 0.0)
            for a in range(16):
                k = (d * 16 + a) % 8
                parts[k] = parts[k] + cd[a * 8:(a + 1) * 8, :]
        cacc = cacc + jnp.where(lc == 1, 1.0, 0.0)
        return tuple(parts) + (cacc,)

    out = lax.fori_loop(0, NTILES, step,
                        tuple(jnp.zeros((8, 128), jnp.float32)
                              for _ in range(8))
                        + (jnp.zeros((ROWS_PER_TILE, 1), jnp.float32),))
    parts, cacc = out[:8], out[8]
    sum_ref[...] = (((parts[0] + parts[1]) + (parts[2] + parts[3]))
                    + ((parts[4] + parts[5]) + (parts[6] + parts[7])))
    cnt_ref[...] = cacc


_pairwise_tc = pl.pallas_call(
    _tc_body,
    out_shape=[
        jax.ShapeDtypeStruct((8, 128), jnp.float32),
        jax.ShapeDtypeStruct((ROWS_PER_TILE, 1), jnp.float32),
    ],
)


@jax.jit
def kernel(prediction, label):
    pcol = prediction.reshape(N, 1)
    lcol = label.reshape(N, 1)
    prow = prediction.reshape(1, N)
    lrow = label.reshape(1, N)
    sums, cnts = _pairwise_tc(pcol, lcol, prow, lrow)
    loss_sum = jnp.sum(sums)
    n1 = jnp.sum(cnts)
    n0 = jnp.float32(N) - n1
    count = n1 * n0
    return jnp.where(count > 0, loss_sum / count, jnp.float32(0.0))


# speedup vs baseline: 1.8440x; 1.0760x over previous
import jax
import jax.numpy as jnp
from jax import lax
from jax.experimental import pallas as pl
from jax.experimental.pallas import tpu as pltpu
from jax.experimental.pallas import tpu_sc as plsc

N = 4096
L = 16
NSUB = 32
CHUNK = N // NSUB
QV = 4
NBLK = N // (QV * L)
NEG_BIG = -1e30


def _sc_body(p_hbm, lab_hbm, sums_hbm, meta_hbm,
             pv, labv, qbuf, rsm, obuf, mbuf):
    c = lax.axis_index("c")
    s = lax.axis_index("s")
    wid = c * 16 + s
    base = wid * CHUNK

    pltpu.sync_copy(p_hbm, pv.at[pl.ds(0, N)])
    pltpu.sync_copy(lab_hbm, labv.at[pl.ds(0, N)])

    neg = jnp.full((L,), NEG_BIG, dtype=jnp.float32)

    def build(jv, carry):
        v = pv[pl.ds(jv * L, L)]
        m = labv[pl.ds(jv * L, L)] == 0
        qbuf[pl.ds(jv * L, L)] = jnp.where(m, v, neg)
        return carry

    lax.fori_loop(0, N // L, build, jnp.int32(0), unroll=4)

    def rowc(g, cnt):
        lv = labv[pl.ds(base + g * L, L)]
        pvv = pv[pl.ds(base + g * L, L)]
        for k in range(L):
            li = lv[k]
            pi = pvv[k]

            @pl.when(li == 1)
            def _(cnt=cnt, pi=pi):
                rsm[cnt] = pi

            cnt = cnt + jnp.where(li == 1, jnp.int32(1), jnp.int32(0))
        return cnt

    n1w = lax.fori_loop(0, CHUNK // L, rowc, jnp.int32(0))

    rsm[n1w] = jnp.float32(-NEG_BIG)
    rsm[n1w + 1] = jnp.float32(-NEG_BIG)
    n_pair = (n1w + 1) // 2

    zero = jnp.zeros((L,), dtype=jnp.float32)

    def qblock(b, gaccs):
        q0 = qbuf[pl.ds(b * (QV * L), L)]
        q1 = qbuf[pl.ds(b * (QV * L) + L, L)]
        q2 = qbuf[pl.ds(b * (QV * L) + 2 * L, L)]
        q3 = qbuf[pl.ds(b * (QV * L) + 3 * L, L)]

        def rows(i, accs):
            b0, b1, b2, b3 = accs
            pi = rsm[2 * i]
            pj = rsm[2 * i + 1]
            b0 = b0 + jnp.maximum(q0 - pi, 0.0)
            b1 = b1 + jnp.maximum(q1 - pi, 0.0)
            b2 = b2 + jnp.maximum(q2 - pi, 0.0)
            b3 = b3 + jnp.maximum(q3 - pi, 0.0)
            b0 = b0 + jnp.maximum(q0 - pj, 0.0)
            b1 = b1 + jnp.maximum(q1 - pj, 0.0)
            b2 = b2 + jnp.maximum(q2 - pj, 0.0)
            b3 = b3 + jnp.maximum(q3 - pj, 0.0)
            return (b0, b1, b2, b3)

        return lax.fori_loop(0, n_pair, rows, gaccs)

    a0, a1, a2, a3 = lax.fori_loop(
        0, NBLK, qblock, (zero, zero, zero, zero))

    obuf[...] = (a0 + a1) + (a2 + a3)
    pltpu.sync_copy(obuf, sums_hbm.at[wid])

    lane = lax.broadcasted_iota(jnp.int32, (L,), 0)
    n0w = jnp.int32(CHUNK) - n1w
    meta = jnp.where(lane == 0, n1w.astype(jnp.float32),
                     jnp.where(lane == 1, n0w.astype(jnp.float32), 0.0))
    mbuf[...] = meta
    pltpu.sync_copy(mbuf, meta_hbm.at[wid])


_mesh = plsc.VectorSubcoreMesh(core_axis_name="c", subcore_axis_name="s")

_pairwise_sc = pl.kernel(
    _sc_body,
    out_type=(jax.ShapeDtypeStruct((NSUB, L), jnp.float32),
              jax.ShapeDtypeStruct((NSUB, L), jnp.float32)),
    mesh=_mesh,
    scratch_types=[
        pltpu.VMEM((N + L,), jnp.float32),
        pltpu.VMEM((N + L,), jnp.int32),
        pltpu.VMEM((N,), jnp.float32),
        pltpu.SMEM((CHUNK + 2,), jnp.float32),
        pltpu.VMEM((L,), jnp.float32),
        pltpu.VMEM((L,), jnp.float32),
    ],
)


ROWS_PER_TILE = 128
NTILES = N // ROWS_PER_TILE


def _tc_body(rc_ref, q_ref, out_ref):
    q = q_ref[...]

    def step(i, carry):
        parts = list(carry[:8])
        cacc = carry[8]
        r = rc_ref[pl.ds(i * ROWS_PER_TILE, ROWS_PER_TILE), :]
        rb = jnp.broadcast_to(r, (ROWS_PER_TILE, 128))
        for d in range(N // 128):
            cd = jnp.maximum(q[:, d * 128:(d + 1) * 128] - rb, 0.0)
            for a in range(16):
                k = (d * 16 + a) % 8
                parts[k] = parts[k] + cd[a * 8:(a + 1) * 8, :]
        cacc = cacc + jnp.where(r < jnp.float32(1e29), 1.0, 0.0)
        return tuple(parts) + (cacc,)

    out = lax.fori_loop(0, NTILES, step,
                        tuple(jnp.zeros((8, 128), jnp.float32)
                              for _ in range(8))
                        + (jnp.zeros((ROWS_PER_TILE, 1), jnp.float32),))
    parts, cacc = out[:8], out[8]
    total = (((parts[0] + parts[1]) + (parts[2] + parts[3]))
             + ((parts[4] + parts[5]) + (parts[6] + parts[7])))
    loss_sum = jnp.sum(total)
    n1 = jnp.sum(cacc)
    si = lax.broadcasted_iota(jnp.int32, (8, 128), 0)
    li = lax.broadcasted_iota(jnp.int32, (8, 128), 1)
    out_ref[...] = jnp.where((si == 0) & (li == 0), loss_sum,
                             jnp.where((si == 1) & (li == 0), n1, 0.0))


_pairwise_tc = pl.pallas_call(
    _tc_body,
    out_shape=jax.ShapeDtypeStruct((8, 128), jnp.float32),
)


@jax.jit
def kernel(prediction, label):
    pcol = prediction.reshape(N, 1)
    lcol = label.reshape(N, 1)
    rcol = jnp.where(lcol == 1, pcol, -NEG_BIG)
    qrow = jnp.where(lcol.reshape(1, N) == 0,
                     pcol.reshape(1, N), NEG_BIG)
    out = _pairwise_tc(rcol, qrow)
    loss_sum = out[0, 0]
    n1 = out[1, 0]
    n0 = jnp.float32(N) - n1
    count = n1 * n0
    return jnp.where(count > 0, loss_sum / count, jnp.float32(0.0))
